# baseline (speedup 1.0000x reference)
import functools

import jax
import jax.numpy as jnp
from jax import lax
from jax.experimental import pallas as pl
from jax.experimental.pallas import tpu as pltpu
from jax.experimental.pallas import tpu_sc as plsc

B = 4096
F = 26
U1 = 64
U2 = 32

NC = 2
NS = 16
NW = NC * NS
RPW = B // NW
SUB = 4
G = RPW // SUB
CH = SUB * F


def _sc_embed(idx, vals, table):
  mesh = plsc.VectorSubcoreMesh(core_axis_name="c", subcore_axis_name="s")

  @functools.partial(
      pl.kernel,
      mesh=mesh,
      out_type=jax.ShapeDtypeStruct((B, U1), jnp.float32),
      compiler_params=pltpu.CompilerParams(use_tc_tiling_on_sc=False),
      scratch_types=[
          pltpu.VMEM((G, CH), jnp.int32),
          pltpu.VMEM((RPW, 32), jnp.float32),
          pltpu.VMEM((CH, U1), jnp.float32),
          pltpu.VMEM((RPW, U1), jnp.float32),
          pltpu.SemaphoreType.DMA,
      ],
  )
  def k(idx_hbm, val_hbm, tab_hbm, out_hbm, idx_v, val_v, rows_v, out_v, sem):
    wid = lax.axis_index("s") * NC + lax.axis_index("c")
    base = wid * RPW
    pltpu.sync_copy(idx_hbm.at[wid], idx_v)
    pltpu.sync_copy(val_hbm.at[pl.ds(base, RPW)], val_v)

    def chunk_body(g, carry):
      pltpu.async_copy(tab_hbm.at[idx_v.at[g]], rows_v, sem).wait()

      def row_body(bl, carry2):
        row = g * SUB + bl
        rbase = bl * F
        wv = [val_v[row, pl.ds(h * 16, 16)] for h in range(2)]
        acc = [jnp.zeros((16,), jnp.float32) for _ in range(U1 // 16)]
        for f in range(F):
          w = wv[f // 16][f % 16]
          r = rbase + f
          for j in range(U1 // 16):
            acc[j] = acc[j] + w * rows_v[r, pl.ds(j * 16, 16)]
        for j in range(U1 // 16):
          out_v[row, pl.ds(j * 16, 16)] = acc[j]
        return carry2

      return lax.fori_loop(0, SUB, row_body, carry)

    lax.fori_loop(0, G, chunk_body, 0)
    pltpu.sync_copy(out_v, out_hbm.at[pl.ds(base, RPW)])

  return k(idx, vals, table)


def _tc_tail(x, b1, k2, b2):
  TB = 512

  def body(x_ref, b1_ref, k2_ref, b2_ref, o_ref):
    xb = jnp.maximum(x_ref[...] + b1_ref[...], 0.0)
    y = jnp.dot(xb, k2_ref[...], preferred_element_type=jnp.float32)
    o_ref[...] = jax.nn.sigmoid(y + b2_ref[...])

  return pl.pallas_call(
      body,
      grid=(B // TB,),
      in_specs=[
          pl.BlockSpec((TB, U1), lambda i: (i, 0)),
          pl.BlockSpec((1, U1), lambda i: (0, 0)),
          pl.BlockSpec((U1, U2), lambda i: (0, 0)),
          pl.BlockSpec((1, U2), lambda i: (0, 0)),
      ],
      out_specs=pl.BlockSpec((TB, U2), lambda i: (i, 0)),
      out_shape=jax.ShapeDtypeStruct((B, U2), jnp.float32),
  )(x, b1, k2, b2)


def kernel(indices, values, kernel1, bias1, kernel2, bias2):
  idx = indices.astype(jnp.int32).reshape(NW, G, CH)
  vals = jnp.pad(values, ((0, 0), (0, 32 - F)))
  out1 = _sc_embed(idx, vals, kernel1)
  return _tc_tail(out1, bias1.reshape(1, U1), kernel2, bias2.reshape(1, U2))


# speedup vs baseline: 1.0030x; 1.0030x over previous
import functools

import jax
import jax.numpy as jnp
from jax import lax
from jax.experimental import pallas as pl
from jax.experimental.pallas import tpu as pltpu
from jax.experimental.pallas import tpu_sc as plsc

B = 4096
F = 26
U1 = 64
U2 = 32

NC = 2
NS = 16
NW = NC * NS
RPW = B // NW
SUB = 4
G = RPW // SUB
CH = SUB * F


def _sc_embed(idx, vals, table):
  mesh = plsc.VectorSubcoreMesh(
      core_axis_name="c", subcore_axis_name="s", num_cores=NC, num_subcores=NS)

  @functools.partial(
      pl.kernel,
      mesh=mesh,
      out_type=jax.ShapeDtypeStruct((B, U1), jnp.float32),
      compiler_params=pltpu.CompilerParams(use_tc_tiling_on_sc=False),
      scratch_types=[
          pltpu.VMEM((G, CH), jnp.int32),
          pltpu.VMEM((RPW, 32), jnp.float32),
          pltpu.VMEM((CH, U1), jnp.float32),
          pltpu.VMEM((RPW, U1), jnp.float32),
          pltpu.SemaphoreType.DMA,
      ],
  )
  def k(idx_hbm, val_hbm, tab_hbm, out_hbm, idx_v, val_v, rows_v, out_v, sem):
    wid = lax.axis_index("s") * NC + lax.axis_index("c")
    base = wid * RPW
    pltpu.sync_copy(idx_hbm.at[wid], idx_v)
    pltpu.sync_copy(val_hbm.at[pl.ds(base, RPW)], val_v)

    def chunk_body(g, carry):
      pltpu.async_copy(tab_hbm.at[idx_v.at[g]], rows_v, sem).wait()

      def row_body(bl, carry2):
        row = g * SUB + bl
        rbase = bl * F
        wv = [val_v[row, pl.ds(h * 16, 16)] for h in range(2)]
        acc = [jnp.zeros((16,), jnp.float32) for _ in range(U1 // 16)]
        for f in range(F):
          w = wv[f // 16][f % 16]
          r = rbase + f
          for j in range(U1 // 16):
            acc[j] = acc[j] + w * rows_v[r, pl.ds(j * 16, 16)]
        for j in range(U1 // 16):
          out_v[row, pl.ds(j * 16, 16)] = acc[j]
        return carry2

      return lax.fori_loop(0, SUB, row_body, carry)

    lax.fori_loop(0, G, chunk_body, 0)
    pltpu.sync_copy(out_v, out_hbm.at[pl.ds(base, RPW)])

  return k(idx, vals, table)


def _tc_tail(x, b1, k2, b2):
  TB = 512

  def body(x_ref, b1_ref, k2_ref, b2_ref, o_ref):
    xb = jnp.maximum(x_ref[...] + b1_ref[...], 0.0)
    y = jnp.dot(xb, k2_ref[...], preferred_element_type=jnp.float32)
    o_ref[...] = jax.nn.sigmoid(y + b2_ref[...])

  return pl.pallas_call(
      body,
      grid=(B // TB,),
      in_specs=[
          pl.BlockSpec((TB, U1), lambda i: (i, 0)),
          pl.BlockSpec((1, U1), lambda i: (0, 0)),
          pl.BlockSpec((U1, U2), lambda i: (0, 0)),
          pl.BlockSpec((1, U2), lambda i: (0, 0)),
      ],
      out_specs=pl.BlockSpec((TB, U2), lambda i: (i, 0)),
      out_shape=jax.ShapeDtypeStruct((B, U2), jnp.float32),
  )(x, b1, k2, b2)


def kernel(indices, values, kernel1, bias1, kernel2, bias2):
  idx = indices.astype(jnp.int32).reshape(NW, G, CH)
  vals = jnp.pad(values, ((0, 0), (0, 32 - F)))
  out1 = _sc_embed(idx, vals, kernel1)
  return _tc_tail(out1, bias1.reshape(1, U1), kernel2, bias2.reshape(1, U2))
